# baseline (speedup 1.0000x reference)
import functools

import jax
import jax.numpy as jnp
from jax import lax
from jax.experimental import pallas as pl
from jax.experimental.pallas import tpu as pltpu
from jax.experimental.pallas import tpu_sc as plsc

TOPK = 5
COST = 0.001
N_ASSETS = 1000
BATCH = 16384
ROW_BLOCK = 512

_IDX_BITS = 1023

_N_SEL = 2 * TOPK
_PAD_SEL = 16
_N_WORKERS = 32
_GATHER_TOTAL = BATCH * _PAD_SEL
_PER_WORKER = _GATHER_TOTAL // _N_WORKERS


def _keys_kernel(p_ref, *out_refs):
    p = p_ref[...]
    b = jax.lax.bitcast_convert_type(p, jnp.int32)
    col = jax.lax.broadcasted_iota(jnp.int32, p.shape, 1)
    keyed = (b & jnp.int32(~_IDX_BITS)) | (jnp.int32(_IDX_BITS) - col)
    key = jax.lax.bitcast_convert_type(keyed, jnp.float32)

    ninf = jnp.float32(-jnp.inf)
    pinf = jnp.float32(jnp.inf)

    km = key
    for j in range(TOPK):
        cur = jnp.max(km, axis=1, keepdims=True)
        km = jnp.where(km == cur, ninf, km)
        out_refs[j][...] = cur

    kn = key
    for j in range(TOPK):
        cur = jnp.min(kn, axis=1, keepdims=True)
        kn = jnp.where(kn == cur, pinf, kn)
        out_refs[TOPK + j][...] = cur


def _sc_gather(t_hbm, i_hbm, o_hbm, idx_v, g_v, sem):
    wid = lax.axis_index("s") * 2 + lax.axis_index("c")
    base = wid * _PER_WORKER
    pltpu.sync_copy(i_hbm.at[pl.ds(base, _PER_WORKER)], idx_v)
    pltpu.async_copy(t_hbm.at[idx_v], g_v, sem).wait()
    pltpu.sync_copy(g_v, o_hbm.at[pl.ds(base, _PER_WORKER)])


def _sharpe_kernel(g_ref, w_ref, o_ref):
    g = g_ref[...]
    w = w_ref[...]
    ret = jnp.sum(w * g, axis=1) - COST * jnp.sum(jnp.abs(w), axis=1)
    n = BATCH
    mean = jnp.sum(ret) / n
    var = jnp.sum((ret - mean) ** 2) / (n - 1)
    std = jnp.sqrt(var) + 1e-8
    o_ref[...] = jnp.full((1, 1), -(mean / std), dtype=jnp.float32)


@jax.jit
def kernel(predictions, targets):
    n_blocks = BATCH // ROW_BLOCK
    t_flat = targets.reshape(BATCH * N_ASSETS)
    col_spec = pl.BlockSpec((ROW_BLOCK, 1), lambda i: (i, 0))
    keys = pl.pallas_call(
        _keys_kernel,
        grid=(n_blocks,),
        in_specs=[pl.BlockSpec((ROW_BLOCK, N_ASSETS), lambda i: (i, 0))],
        out_specs=[col_spec] * _N_SEL,
        out_shape=[jax.ShapeDtypeStruct((BATCH, 1), jnp.float32)] * _N_SEL,
        compiler_params=pltpu.CompilerParams(
            dimension_semantics=("parallel",),
        ),
    )(predictions)

    kcat = jnp.concatenate(keys, axis=1)
    cols = _IDX_BITS - (
        jax.lax.bitcast_convert_type(kcat, jnp.int32) & _IDX_BITS
    )
    tcols, bcols = cols[:, :TOPK], cols[:, TOPK:]
    overlap = (tcols[:, :, None] == bcols[:, None, :]).any(axis=2)
    inv_k = jnp.float32(1.0 / TOPK)
    wt = jnp.where(overlap, 0.0, inv_k)
    w = jnp.concatenate(
        [wt, jnp.full((BATCH, TOPK), -inv_k, jnp.float32),
         jnp.zeros((BATCH, _PAD_SEL - _N_SEL), jnp.float32)], axis=1)
    rows = jax.lax.broadcasted_iota(jnp.int32, (BATCH, _PAD_SEL), 0)
    idx = jnp.concatenate(
        [cols, jnp.zeros((BATCH, _PAD_SEL - _N_SEL), jnp.int32)], axis=1)
    flat_idx = (rows * N_ASSETS + idx).reshape(_GATHER_TOTAL)

    sc_mesh = plsc.VectorSubcoreMesh(core_axis_name="c", subcore_axis_name="s")
    gather = pl.kernel(
        _sc_gather,
        mesh=sc_mesh,
        out_type=jax.ShapeDtypeStruct((_GATHER_TOTAL,), jnp.float32),
        scratch_types=[
            pltpu.VMEM((_PER_WORKER,), jnp.int32),
            pltpu.VMEM((_PER_WORKER,), jnp.float32),
            pltpu.SemaphoreType.DMA,
        ],
    )
    g = gather(t_flat, flat_idx)

    out = pl.pallas_call(
        _sharpe_kernel,
        out_shape=jax.ShapeDtypeStruct((1, 1), jnp.float32),
    )(g.reshape(BATCH, _PAD_SEL), w)
    return out[0, 0]


# speedup vs baseline: 1.6120x; 1.6120x over previous
import jax
import jax.numpy as jnp
from jax.experimental import pallas as pl
from jax.experimental.pallas import tpu as pltpu

TOPK = 5
COST = 0.001
N_ASSETS = 1000
BATCH = 16384
ROW_BLOCK = 512
N_BLOCKS = BATCH // ROW_BLOCK

_IDX_BITS = 1023


def _rows_kernel(p_ref, t_ref, part_ref):
    p = p_ref[...]
    t = t_ref[...]
    b = jax.lax.bitcast_convert_type(p, jnp.int32)
    col = jax.lax.broadcasted_iota(jnp.int32, p.shape, 1)
    keyed = (b & jnp.int32(~_IDX_BITS)) | (jnp.int32(_IDX_BITS) - col)
    key = jax.lax.bitcast_convert_type(keyed, jnp.float32)

    ninf = jnp.float32(-jnp.inf)
    pinf = jnp.float32(jnp.inf)

    km = key
    for _ in range(TOPK):
        cur = jnp.max(km, axis=1, keepdims=True)
        km = jnp.where(km == cur, ninf, km)
    topmask = km == ninf

    kn = key
    for _ in range(TOPK):
        cur = jnp.min(kn, axis=1, keepdims=True)
        kn = jnp.where(kn == cur, pinf, kn)
    botmask = kn == pinf

    inv_k = jnp.float32(1.0 / TOPK)
    contrib = jnp.where(botmask, -t, jnp.where(topmask, t, 0.0))
    gross = inv_k * jnp.sum(contrib, axis=1)
    overlap = jnp.sum((topmask & botmask).astype(jnp.float32), axis=1)
    ret = gross - COST * inv_k * (2.0 * TOPK - overlap)
    s1 = jnp.sum(ret)
    s2 = jnp.sum(ret * ret)
    part_ref[...] = jnp.concatenate(
        [s1[None, None], s2[None, None]], axis=1)[None]


def _sharpe_kernel(part_ref, o_ref):
    s1 = jnp.sum(part_ref[:, 0, 0])
    s2 = jnp.sum(part_ref[:, 0, 1])
    n = BATCH
    mean = s1 / n
    var = (s2 - n * mean * mean) / (n - 1)
    std = jnp.sqrt(var) + 1e-8
    o_ref[...] = jnp.full((1, 1), -(mean / std), dtype=jnp.float32)


@jax.jit
def kernel(predictions, targets):
    parts = pl.pallas_call(
        _rows_kernel,
        grid=(N_BLOCKS,),
        in_specs=[
            pl.BlockSpec((ROW_BLOCK, N_ASSETS), lambda i: (i, 0)),
            pl.BlockSpec((ROW_BLOCK, N_ASSETS), lambda i: (i, 0)),
        ],
        out_specs=pl.BlockSpec((1, 1, 2), lambda i: (i, 0, 0)),
        out_shape=jax.ShapeDtypeStruct((N_BLOCKS, 1, 2), jnp.float32),
        compiler_params=pltpu.CompilerParams(
            dimension_semantics=("parallel",),
        ),
    )(predictions, targets)

    out = pl.pallas_call(
        _sharpe_kernel,
        out_shape=jax.ShapeDtypeStruct((1, 1), jnp.float32),
    )(parts)
    return out[0, 0]
